# baseline (speedup 1.0000x reference)
import functools

import numpy as np
import jax
import jax.numpy as jnp
from jax import lax
from jax.experimental import pallas as pl
from jax.experimental.pallas import tpu as pltpu
from jax.experimental.pallas import tpu_sc as plsc

NC, NS = 2, 16
NW = NC * NS
D = 128
B, S, K = 4096, 200, 50
R = B * K
C = 100
NSLAB = R // C
BB = NSLAB // NW
M = 8
P = 4
NR = BB // M


def _make_idx():
    r = np.arange(R, dtype=np.int64)
    idx = (r // K) * S + (r % K) * 4
    return idx.reshape(NW, BB, C).astype(np.int32)


_IDX = _make_idx()

_mesh = plsc.VectorSubcoreMesh(core_axis_name="c", subcore_axis_name="s")


@functools.partial(
    pl.kernel,
    out_type=jax.ShapeDtypeStruct((NSLAB, C, D), jnp.float32),
    mesh=_mesh,
    scratch_types=[
        pltpu.VMEM((BB, C), jnp.int32),
        [pltpu.VMEM((C, D), jnp.float32)] * M,
        [pltpu.SemaphoreType.DMA] * M,
        [pltpu.SemaphoreType.DMA] * M,
    ],
)
def _gather_sc(x_hbm, idx_hbm, out_hbm, idx_v, bufs, gsems, ssems):
    c = lax.axis_index("c")
    s = lax.axis_index("s")
    wid = c * NS + s
    base = wid * BB
    pltpu.sync_copy(idx_hbm.at[wid], idx_v)

    for b in range(P):
        pltpu.async_copy(x_hbm.at[idx_v.at[b]], bufs[b], gsems[b])

    @pl.loop(0, NR)
    def _round(r):
        for b in range(M):
            j = r * M + b
            pltpu.make_async_copy(x_hbm.at[idx_v.at[j]], bufs[b], gsems[b]).wait()
            pltpu.async_copy(bufs[b], out_hbm.at[base + j], ssems[b])

            bs = (b - P) % M

            @pl.when(j >= P)
            def _drain():
                pltpu.make_async_copy(
                    bufs[bs], out_hbm.at[base + j - P], ssems[bs]
                ).wait()

            bn = (b + P) % M

            @pl.when(j + P < BB)
            def _refill():
                pltpu.async_copy(x_hbm.at[idx_v.at[j + P]], bufs[bn], gsems[bn])

    for b in range(P):
        j = BB - P + b
        pltpu.make_async_copy(
            bufs[j % M], out_hbm.at[base + j], ssems[j % M]
        ).wait()


def kernel(x):
    x2 = x.reshape(B * S, D)
    out3 = _gather_sc(x2, _IDX)
    return out3.reshape(B, K, D)


# speedup vs baseline: 1.7888x; 1.7888x over previous
import functools

import numpy as np
import jax
import jax.numpy as jnp
from jax import lax
from jax.experimental import pallas as pl
from jax.experimental.pallas import tpu as pltpu
from jax.experimental.pallas import tpu_sc as plsc

NC, NS = 2, 16
NW = NC * NS
D = 128
B, S, K = 4096, 200, 50
R = B * K
C = 50
NSLAB = R // C
BB = NSLAB // NW
M = 8
P = 4
NR = BB // M


def _make_idx():
    r = np.arange(R, dtype=np.int64)
    idx = (r // K) * S + (r % K) * 4
    return idx.reshape(NW, BB, C).astype(np.int32)


_IDX = _make_idx()

_mesh = plsc.VectorSubcoreMesh(core_axis_name="c", subcore_axis_name="s")


@functools.partial(
    pl.kernel,
    out_type=jax.ShapeDtypeStruct((NSLAB, C, D), jnp.float32),
    mesh=_mesh,
    scratch_types=[
        pltpu.VMEM((BB, C), jnp.int32),
        [pltpu.VMEM((C, D), jnp.float32)] * M,
        [pltpu.SemaphoreType.DMA] * M,
        [pltpu.SemaphoreType.DMA] * M,
    ],
)
def _gather_sc(x_hbm, idx_hbm, out_hbm, idx_v, bufs, gsems, ssems):
    c = lax.axis_index("c")
    s = lax.axis_index("s")
    wid = c * NS + s
    base = wid * BB
    pltpu.sync_copy(idx_hbm.at[wid], idx_v)

    for b in range(P):
        pltpu.async_copy(x_hbm.at[idx_v.at[b]], bufs[b], gsems[b])

    @pl.loop(0, NR)
    def _round(r):
        for b in range(M):
            j = r * M + b
            pltpu.make_async_copy(x_hbm.at[idx_v.at[j]], bufs[b], gsems[b]).wait()
            pltpu.async_copy(bufs[b], out_hbm.at[base + j], ssems[b])

            bs = (b - P) % M

            @pl.when(j >= P)
            def _drain():
                pltpu.make_async_copy(
                    bufs[bs], out_hbm.at[base + j - P], ssems[bs]
                ).wait()

            bn = (b + P) % M

            @pl.when(j + P < BB)
            def _refill():
                pltpu.async_copy(x_hbm.at[idx_v.at[j + P]], bufs[bn], gsems[bn])

    for b in range(P):
        j = BB - P + b
        pltpu.make_async_copy(
            bufs[j % M], out_hbm.at[base + j], ssems[j % M]
        ).wait()


def kernel(x):
    x2 = x.reshape(B * S, D)
    return _gather_sc(x2, _IDX)
